# Initial kernel scaffold; baseline (speedup 1.0000x reference)
import jax
import jax.numpy as jnp
from jax.experimental import pallas as pl


def kernel(cell_id, tissue_id, src_c2t, dst_c2t, src_t2c, dst_t2c, w_c2t, w_t2c, embed_feat, embed_tissue, W_t, b_t, W_c, b_c, gn_t_g, gn_t_b, gn_c_g, gn_c_b, Ws_c2t, Wn_c2t, b_c2t, Ws_t2c, Wn_t2c, b_t2c, gn1_g, gn1_b):
    raise NotImplementedError("write your pallas kernel here")


# speedup vs baseline: 7.3551x; 7.3551x over previous
import jax
import jax.numpy as jnp
from jax import lax
from jax.experimental import pallas as pl
from jax.experimental.pallas import tpu as pltpu
from jax.experimental.pallas import tpu_sc as plsc

N_CELL = 100000
N_TISSUE = 1000
HID = 128
FEAT = 2048
E = 300000

ECHUNK = 1024
NCHUNKS_E = 294
EP = NCHUNKS_E * ECHUNK
PAD_DST_C2T = 1001
PAD_DST_T2C = 100008

CSTRIDE = 2052
HALF = 500
JUNK_ROW = HALF
C_HALF_WORDS = 1028096
ACC_BASE = C_HALF_WORDS
ACC_WORDS = 400128
SPMEM_WORDS = 1433600
TILE_SPW = SPMEM_WORDS // 16
ZCHUNK = 11200

NP = 102400
NCHUNKS_N = NP // 128

_mesh = plsc.VectorSubcoreMesh(
    core_axis_name="c", subcore_axis_name="s", num_cores=2, num_subcores=16)


def _gelu(x):
    return 0.5 * x * (1.0 + lax.erf(x * 0.7071067811865476))


def _gn_rows(x, g, b, eps=1e-5):
    n, c = x.shape
    gi = lax.broadcasted_iota(jnp.int32, (c, c), 0) // 32
    gj = lax.broadcasted_iota(jnp.int32, (c, c), 1) // 32
    M = jnp.where(gi == gj, 1.0 / 32.0, 0.0).astype(jnp.float32)
    mu = lax.dot_general(x, M, (((1,), (0,)), ((), ())),
                         preferred_element_type=jnp.float32)
    ex2 = lax.dot_general(x * x, M, (((1,), (0,)), ((), ())),
                          preferred_element_type=jnp.float32)
    var = ex2 - mu * mu
    return (x - mu) * lax.rsqrt(var + eps) * g + b


def _lrelu(x):
    return jnp.where(x >= 0, x, 0.01 * x)


def _mm_t(a, w):
    return lax.dot_general(a, w, (((1,), (1,)), ((), ())),
                           preferred_element_type=jnp.float32)


def _p1_body(ef, et, W_c, b_c, gnc_g, gnc_b, W_t, b_t, gnt_g, gnt_b,
             Ws_t2c, b_t2c, Wn_c2t, Ws_c2t, b_c2t, Wn_t2c,
             cself, cWn, tself, v01):
    ctab = _gn_rows(_gelu(_mm_t(_lrelu(ef[...]), W_c[...]) + b_c[...]),
                    gnc_g[...], gnc_b[...])
    cself[...] = _mm_t(ctab, Ws_t2c[...]) + b_t2c[...]
    cWn[...] = _mm_t(ctab, Wn_c2t[...])
    ttab = _gn_rows(_gelu(_mm_t(_lrelu(et[...]), W_t[...]) + b_t[...]),
                    gnt_g[...], gnt_b[...])
    tself[...] = _mm_t(ttab, Ws_c2t[...]) + b_c2t[...]
    v01[...] = _mm_t(ttab, Wn_t2c[...])


def _p1(ef, et, W_c, b_c, gnc_g, gnc_b, W_t, b_t, gnt_g, gnt_b,
        Ws_t2c, b_t2c, Wn_c2t, Ws_c2t, b_c2t, Wn_t2c):
    return pl.pallas_call(
        _p1_body,
        out_shape=(
            jax.ShapeDtypeStruct((FEAT, HID), jnp.float32),
            jax.ShapeDtypeStruct((FEAT, HID), jnp.float32),
            jax.ShapeDtypeStruct((8, HID), jnp.float32),
            jax.ShapeDtypeStruct((8, HID), jnp.float32),
        ),
    )(ef, et, W_c, b_c, gnc_g, gnc_b, W_t, b_t, gnt_g, gnt_b,
      Ws_t2c, b_t2c, Wn_c2t, Ws_c2t, b_c2t, Wn_t2c)


def _p2_body(srcc, dstc, wc, srct, dstt, wt, cellid, tisid,
             out_sp,
             tis_v, srcv, dstv, wv, cidv, fidx1, fidx2, onesv, zbuf, sem,
             spmem):
    core = lax.axis_index("c")
    sub = lax.axis_index("s")

    def fill_z(i, _):
        zbuf[pl.ds(i * 16, 16)] = jnp.zeros((16,), jnp.float32)
        return 0
    lax.fori_loop(0, ZCHUNK // 16, fill_z, 0)

    def fill_o(i, _):
        onesv[pl.ds(i * 16, 16)] = jnp.ones((16,), jnp.float32)
        return 0
    lax.fori_loop(0, ECHUNK // 16, fill_o, 0)

    zb = sub * TILE_SPW
    for j in range(8):
        pltpu.sync_copy(zbuf, spmem.at[pl.ds(zb + j * ZCHUNK, ZCHUNK)])
    pltpu.sync_copy(tisid, tis_v)
    plsc.subcore_barrier()

    row_lo = core * HALF

    def chunk_c(i, _):
        ch = sub + i * 16

        @pl.when(ch < NCHUNKS_E)
        def _():
            off = ch * ECHUNK
            pltpu.sync_copy(srcc.at[pl.ds(off, ECHUNK)], srcv)
            pltpu.sync_copy(dstc.at[pl.ds(off, ECHUNK)], dstv)
            pltpu.sync_copy(wc.at[pl.ds(off, ECHUNK)], wv)
            pltpu.async_copy(cellid.at[srcv], cidv, sem).wait()

            def vec(k, _):
                d16 = dstv[pl.ds(k * 16, 16)]
                c16 = cidv[pl.ds(k * 16, 16)]
                loc = d16 - row_lo
                ok = jnp.logical_and(loc >= 0, loc < HALF)
                row = jnp.where(ok, loc, JUNK_ROW)
                fidx1[pl.ds(k * 16, 16)] = row * CSTRIDE + c16
                fidx2[pl.ds(k * 16, 16)] = row * CSTRIDE + 2048
                return 0
            lax.fori_loop(0, ECHUNK // 16, vec, 0)
            pltpu.sync_copy(wv, spmem.at[fidx1], add=True)
            pltpu.sync_copy(onesv, spmem.at[fidx2], add=True)
        return 0
    lax.fori_loop(0, 19, chunk_c, 0)

    def chunk_t(i, _):
        local = sub + i * 16

        @pl.when(local < NCHUNKS_E // 2)
        def _():
            ch = core * (NCHUNKS_E // 2) + local
            off = ch * ECHUNK
            pltpu.sync_copy(srct.at[pl.ds(off, ECHUNK)], srcv)
            pltpu.sync_copy(dstt.at[pl.ds(off, ECHUNK)], dstv)
            pltpu.sync_copy(wt.at[pl.ds(off, ECHUNK)], wv)

            def vec(k, _):
                s16 = srcv[pl.ds(k * 16, 16)]
                d16 = dstv[pl.ds(k * 16, 16)]
                t16 = plsc.load_gather(tis_v, [s16])
                fidx1[pl.ds(k * 16, 16)] = ACC_BASE + d16 * 4 + t16
                fidx2[pl.ds(k * 16, 16)] = ACC_BASE + d16 * 4 + 2
                return 0
            lax.fori_loop(0, ECHUNK // 16, vec, 0)
            pltpu.sync_copy(wv, spmem.at[fidx1], add=True)
            pltpu.sync_copy(onesv, spmem.at[fidx2], add=True)
        return 0
    lax.fori_loop(0, 10, chunk_t, 0)

    plsc.subcore_barrier()

    for j in range(8):
        off = sub * TILE_SPW + j * ZCHUNK
        pltpu.sync_copy(spmem.at[pl.ds(off, ZCHUNK)], zbuf)
        pltpu.sync_copy(zbuf, out_sp.at[pl.ds(core * SPMEM_WORDS + off,
                                              ZCHUNK)])


def _p2(srcc, dstc, wc, srct, dstt, wt, cellid, tisid):
    return pl.kernel(
        _p2_body,
        out_type=jax.ShapeDtypeStruct((2 * SPMEM_WORDS,), jnp.float32),
        mesh=_mesh,
        scratch_types=(
            pltpu.VMEM((N_TISSUE,), jnp.int32),
            pltpu.VMEM((ECHUNK,), jnp.int32),
            pltpu.VMEM((ECHUNK,), jnp.int32),
            pltpu.VMEM((ECHUNK,), jnp.float32),
            pltpu.VMEM((ECHUNK,), jnp.int32),
            pltpu.VMEM((ECHUNK,), jnp.int32),
            pltpu.VMEM((ECHUNK,), jnp.int32),
            pltpu.VMEM((ECHUNK,), jnp.float32),
            pltpu.VMEM((ZCHUNK,), jnp.float32),
            pltpu.SemaphoreType.DMA,
            pltpu.VMEM_SHARED((SPMEM_WORDS,), jnp.float32),
        ),
        compiler_params=pltpu.CompilerParams(needs_layout_passes=False),
    )(srcc, dstc, wc, srct, dstt, wt, cellid, tisid)


def _p3_body(cid_hbm, acc0_hbm, acc1_hbm, cself_hbm, v01_hbm, x_hbm,
             cidv, acc0v, acc1v, accs, v01v, urows, xbuf, sem):
    core = lax.axis_index("c")
    sub = lax.axis_index("s")
    wid = sub * 2 + core

    pltpu.sync_copy(v01_hbm, v01v)

    def chunk(i, _):
        ch = wid + i * 32
        base = ch * 128
        pltpu.sync_copy(cid_hbm.at[pl.ds(base, 128)], cidv)
        pltpu.sync_copy(acc0_hbm.at[pl.ds(base * 4, 512)], acc0v)
        pltpu.sync_copy(acc1_hbm.at[pl.ds(base * 4, 512)], acc1v)
        pltpu.async_copy(cself_hbm.at[cidv], urows, sem).wait()

        def addacc(k, _):
            accs[pl.ds(k * 16, 16)] = (acc0v[pl.ds(k * 16, 16)]
                                       + acc1v[pl.ds(k * 16, 16)])
            return 0
        lax.fori_loop(0, 32, addacc, 0)

        def row(r, _):
            a_b = plsc.load_gather(accs, [jnp.full((16,), 4 * r, jnp.int32)])
            b_b = plsc.load_gather(accs,
                                   [jnp.full((16,), 4 * r + 1, jnp.int32)])
            d_b = plsc.load_gather(accs,
                                   [jnp.full((16,), 4 * r + 2, jnp.int32)])
            dmax = jnp.maximum(d_b, 1.0)
            s = a_b / dmax
            t = b_b / dmax
            for j in range(8):
                u = urows[r, pl.ds(16 * j, 16)]
                xj = u + s * v01v[0, pl.ds(16 * j, 16)] \
                       + t * v01v[1, pl.ds(16 * j, 16)]
                xbuf[r, pl.ds(16 * j, 16)] = xj
            return 0
        lax.fori_loop(0, 128, row, 0)
        pltpu.sync_copy(xbuf, x_hbm.at[pl.ds(base, 128)])
        return 0
    lax.fori_loop(0, NCHUNKS_N // 32, chunk, 0)


def _p3(cid_p, acc0_p, acc1_p, cself, v01):
    return pl.kernel(
        _p3_body,
        out_type=jax.ShapeDtypeStruct((NP, HID), jnp.float32),
        mesh=_mesh,
        scratch_types=(
            pltpu.VMEM((128,), jnp.int32),
            pltpu.VMEM((512,), jnp.float32),
            pltpu.VMEM((512,), jnp.float32),
            pltpu.VMEM((528,), jnp.float32),
            pltpu.VMEM((2, HID), jnp.float32),
            pltpu.VMEM((128, HID), jnp.float32),
            pltpu.VMEM((128, HID), jnp.float32),
            pltpu.SemaphoreType.DMA,
        ),
        compiler_params=pltpu.CompilerParams(needs_layout_passes=False),
    )(cid_p, acc0_p, acc1_p, cself, v01)


def _p4_body(x, g, b, out):
    out[...] = _gelu(_gn_rows(x[...], g[...], b[...]))


def _p4(x, g, b):
    nblk = NP // 2048
    return pl.pallas_call(
        _p4_body,
        grid=(nblk,),
        in_specs=[
            pl.BlockSpec((2048, HID), lambda i: (i, 0)),
            pl.BlockSpec((1, HID), lambda i: (0, 0)),
            pl.BlockSpec((1, HID), lambda i: (0, 0)),
        ],
        out_specs=pl.BlockSpec((2048, HID), lambda i: (i, 0)),
        out_shape=jax.ShapeDtypeStruct((NP, HID), jnp.float32),
    )(x, g, b)


def _p5_body(c2, cWn, tself, tidb, g, b, out):
    C = c2[...]
    neigh = lax.dot_general(C[:, :2048], cWn[...], (((1,), (0,)), ((), ())),
                            preferred_element_type=jnp.float32)
    deg = C[:, 2048:2049]
    invd = 1.0 / jnp.maximum(deg, 1.0)
    t0 = tself[0:1, :]
    t1 = tself[1:2, :]
    tsel = t0 + tidb[...] * (t1 - t0)
    y = tsel + neigh * invd
    out[...] = _gelu(_gn_rows(y, g[...], b[...]))


def _p5(c2, cWn, tself, tidb, g, b):
    return pl.pallas_call(
        _p5_body,
        out_shape=jax.ShapeDtypeStruct((N_TISSUE, HID), jnp.float32),
    )(c2, cWn, tself, tidb, g, b)


def kernel(cell_id, tissue_id, src_c2t, dst_c2t, src_t2c, dst_t2c,
           w_c2t, w_t2c, embed_feat, embed_tissue, W_t, b_t, W_c, b_c,
           gn_t_g, gn_t_b, gn_c_g, gn_c_b, Ws_c2t, Wn_c2t, b_c2t,
           Ws_t2c, Wn_t2c, b_t2c, gn1_g, gn1_b):
    i32 = jnp.int32
    f32 = jnp.float32
    row = lambda v: v.reshape(1, HID).astype(f32)

    et_p = jnp.zeros((8, HID), f32).at[0:2, :].set(embed_tissue)
    cself, cWn, tself, v01 = _p1(
        embed_feat.astype(f32), et_p, W_c, row(b_c), row(gn_c_g), row(gn_c_b),
        W_t, row(b_t), row(gn_t_g), row(gn_t_b),
        Ws_t2c, row(b_t2c), Wn_c2t, Ws_c2t, row(b_c2t), Wn_t2c)

    padi = lambda v, val: jnp.concatenate(
        [v.astype(i32), jnp.full((EP - E,), val, i32)])
    padf = lambda v: jnp.concatenate([v, jnp.zeros((EP - E,), f32)])
    sp = _p2(
        padi(src_c2t, 0), padi(dst_c2t, PAD_DST_C2T), padf(w_c2t),
        padi(src_t2c, 0), padi(dst_t2c, PAD_DST_T2C), padf(w_t2c),
        cell_id.astype(i32), tissue_id.astype(i32))

    cid_p = jnp.concatenate([cell_id.astype(i32),
                             jnp.zeros((NP - N_CELL,), i32)])
    zpad = jnp.zeros((NP * 4 - N_CELL * 4,), f32)
    acc0_p = jnp.concatenate(
        [lax.dynamic_slice(sp, (ACC_BASE,), (N_CELL * 4,)), zpad])
    acc1_p = jnp.concatenate(
        [lax.dynamic_slice(sp, (SPMEM_WORDS + ACC_BASE,), (N_CELL * 4,)),
         zpad])
    x = _p3(cid_p, acc0_p, acc1_p, cself, v01[0:2, :])

    c_out = _p4(x, gn1_g.reshape(1, HID), gn1_b.reshape(1, HID))

    c0 = lax.dynamic_slice(sp, (0,), (HALF * CSTRIDE,))
    c1 = lax.dynamic_slice(sp, (SPMEM_WORDS,), (HALF * CSTRIDE,))
    c2 = jnp.concatenate([c0.reshape(HALF, CSTRIDE),
                          c1.reshape(HALF, CSTRIDE)], axis=0)
    tidb = jnp.broadcast_to(
        tissue_id.astype(f32).reshape(N_TISSUE, 1), (N_TISSUE, HID))
    t_out = _p5(c2, cWn, tself, tidb,
                gn1_g.reshape(1, HID), gn1_b.reshape(1, HID))

    return jnp.concatenate([c_out[:N_CELL], t_out], axis=0)
